# baseline (speedup 1.0000x reference)
import functools

import jax
import jax.numpy as jnp
from jax import lax
from jax.experimental import pallas as pl
from jax.experimental.pallas import tpu as pltpu
from jax.experimental.pallas import tpu_sc as plsc

N = 10000
RPAD = 10240
NFEAT = 128
NHID = 64
RELL = 86
B = 16384
DPAD = 128

BM_A = 192
GRID_A = -(-N // BM_A)
BM = 256
GRID = RPAD // BM

F32 = jnp.float32
BF16 = jnp.bfloat16
F8 = jnp.float8_e4m3fn
SCALE = 4096.0
SCALE_RHS = 16.0
INV_SCALE = 1.0 / (SCALE * SCALE_RHS)


def _prep_body(x_ref, ws12_ref, wo1_ref, wd1t_ref, wd1b_ref, wd2_ref,
               bd1_ref, bd2_ref,
               a1_ref, a23_ref, wdt_ref, wdb_ref, bconst_ref):
    x = x_ref[...]
    a1_ref[...] = jnp.dot(x, wo1_ref[...], preferred_element_type=F32)
    a23_ref[...] = jnp.dot(x, ws12_ref[...], preferred_element_type=F32)
    wd2 = wd2_ref[...]
    wdt_ref[...] = jnp.dot(wd1t_ref[...], wd2, preferred_element_type=F32)
    wdb_ref[...] = jnp.dot(wd1b_ref[...], wd2, preferred_element_type=F32)
    bconst_ref[...] = (jnp.dot(bd1_ref[...], wd2, preferred_element_type=F32)
                       + bd2_ref[...])


def _prep(x, ws12, wo1, wd1t, wd1b, wd2p, bd1r, bd2r):
    return pl.pallas_call(
        _prep_body,
        out_shape=[
            jax.ShapeDtypeStruct((N, NHID), F32),
            jax.ShapeDtypeStruct((N, 2 * NHID), F32),
            jax.ShapeDtypeStruct((NHID, DPAD), F32),
            jax.ShapeDtypeStruct((NHID, DPAD), F32),
            jax.ShapeDtypeStruct((1, DPAD), F32),
        ],
    )(x, ws12, wo1, wd1t, wd1b, wd2p, bd1r, bd2r)


def _pass_a_body(o_ref, s_ref, a1_ref, a23_ref, ba_ref, bs1_ref, wc_ref,
                 c_ref, t1_ref, o8_ref, s8_ref):
    o = o_ref[...]
    s = s_ref[...]
    o8_ref[...] = (o * SCALE).astype(F8)
    s8_ref[...] = (s * SCALE).astype(F8)
    p = jnp.dot(o.astype(BF16), a1_ref[...].astype(BF16),
                preferred_element_type=F32)
    q = jnp.dot(s.astype(BF16), a23_ref[...].astype(BF16),
                preferred_element_type=F32)
    ox = jnp.maximum(p + q[:, :NHID] + ba_ref[...], 0.0)
    c_ref[...] = jnp.dot(ox, wc_ref[...], preferred_element_type=F32)
    t1_ref[...] = q[:, NHID:] + bs1_ref[...]


def _pass_a(o_adj, s_adj, a1, a23, ba, bs1, wc):
    adj_spec = pl.BlockSpec((BM_A, N), lambda i: (i, 0))
    row_spec = lambda w: pl.BlockSpec((BM_A, w), lambda i: (i, 0))
    full = lambda shape: pl.BlockSpec(shape, lambda i: (0, 0))
    return pl.pallas_call(
        _pass_a_body,
        grid=(GRID_A,),
        in_specs=[adj_spec, adj_spec,
                  full((N, NHID)), full((N, 2 * NHID)),
                  full((1, NHID)), full((1, NHID)), full((NHID, 2 * NHID))],
        out_specs=[row_spec(2 * NHID), row_spec(NHID),
                   adj_spec, adj_spec],
        out_shape=[jax.ShapeDtypeStruct((RPAD, 2 * NHID), F32),
                   jax.ShapeDtypeStruct((RPAD, NHID), F32),
                   jax.ShapeDtypeStruct((RPAD, N), F8),
                   jax.ShapeDtypeStruct((RPAD, N), F8)],
    )(o_adj, s_adj, a1, a23, ba, bs1, wc)


def _pass_b_body(o8_ref, c_ref, t1_ref, bo1s_ref, bo2_ref, ws2o_ref,
                 d_ref, hp_ref):
    c8 = (c_ref[...] * SCALE_RHS).astype(F8)
    r = jnp.dot(o8_ref[...], c8, preferred_element_type=F32) * INV_SCALE
    sx = jnp.maximum(t1_ref[...] + r[:, :NHID] + bo1s_ref[...], 0.0)
    d_ref[...] = jnp.dot(sx, ws2o_ref[...], preferred_element_type=F32)
    hp_ref[...] = r[:, NHID:] + bo2_ref[...]


def _pass_b(o8, c, t1, bo1s, bo2, ws2o):
    full = lambda shape: pl.BlockSpec(shape, lambda i: (0, 0))
    row_spec = lambda w: pl.BlockSpec((BM, w), lambda i: (i, 0))
    return pl.pallas_call(
        _pass_b_body,
        grid=(GRID,),
        in_specs=[pl.BlockSpec((BM, N), lambda i: (i, 0)),
                  full((N, 2 * NHID)),
                  row_spec(NHID),
                  full((1, NHID)), full((1, NHID)), full((NHID, NHID))],
        out_specs=[row_spec(NHID), row_spec(NHID)],
        out_shape=[jax.ShapeDtypeStruct((RPAD, NHID), F32),
                   jax.ShapeDtypeStruct((RPAD, NHID), F32)],
    )(o8, c, t1, bo1s, bo2, ws2o)


def _pass_c_body(s8_ref, d_ref, hp_ref, bs2o_ref, wdt_ref, wdb_ref, bconst_ref,
                 g1_ref, g2_ref):
    d8 = (d_ref[...] * SCALE_RHS).astype(F8)
    s = jnp.dot(s8_ref[...], d8, preferred_element_type=F32) * INV_SCALE
    h = hp_ref[...] + s + bs2o_ref[...]
    g1_ref[...] = jnp.dot(h, wdt_ref[...], preferred_element_type=F32) + bconst_ref[...]
    g2_ref[...] = jnp.dot(h, wdb_ref[...], preferred_element_type=F32)


def _pass_c(s8, d, hp, bs2o, wdt, wdb, bconst):
    full = lambda shape: pl.BlockSpec(shape, lambda i: (0, 0))
    row_spec = lambda w: pl.BlockSpec((BM, w), lambda i: (i, 0))
    return pl.pallas_call(
        _pass_c_body,
        grid=(GRID,),
        in_specs=[pl.BlockSpec((BM, N), lambda i: (i, 0)),
                  full((N, NHID)),
                  row_spec(NHID),
                  full((1, NHID)),
                  full((NHID, DPAD)), full((NHID, DPAD)), full((1, DPAD))],
        out_specs=[row_spec(DPAD), row_spec(DPAD)],
        out_shape=[jax.ShapeDtypeStruct((RPAD, DPAD), F32),
                   jax.ShapeDtypeStruct((RPAD, DPAD), F32)],
    )(s8, d, hp, bs2o, wdt, wdb, bconst)


def _gather_add(g1, g2, idx2):
    info = plsc.get_sparse_core_info()
    nc, ns = info.num_cores, info.num_subcores
    nw = nc * ns
    bpw = B // nw
    chunk = 128
    nch = bpw // chunk
    mesh = plsc.VectorSubcoreMesh(core_axis_name="c", subcore_axis_name="s")

    @functools.partial(
        pl.kernel, mesh=mesh,
        out_type=jax.ShapeDtypeStruct((B, DPAD), F32),
        scratch_types=[
            pltpu.VMEM((nch, chunk), jnp.int32),
            pltpu.VMEM((nch, chunk), jnp.int32),
            pltpu.VMEM((chunk, DPAD), F32),
            pltpu.VMEM((chunk, DPAD), F32),
            pltpu.SemaphoreType.DMA,
        ],
    )
    def k(g1_hbm, g2_hbm, i_hbm, out_hbm, i0_v, i1_v, r1_v, r2_v, sem):
        wid = lax.axis_index("s") * nc + lax.axis_index("c")
        pltpu.sync_copy(i_hbm.at[0, wid], i0_v)
        pltpu.sync_copy(i_hbm.at[1, wid], i1_v)
        nvec = DPAD // 16

        for j in range(nch):
            cp1 = pltpu.async_copy(g1_hbm.at[i0_v.at[j]], r1_v, sem)
            cp2 = pltpu.async_copy(g2_hbm.at[i1_v.at[j]], r2_v, sem)
            cp1.wait()
            cp2.wait()

            def body(r, _):
                for c in range(nvec):
                    sl = pl.ds(c * 16, 16)
                    r1_v[r, sl] = r1_v[r, sl] + r2_v[r, sl]
                return 0

            lax.fori_loop(0, chunk, body, 0)
            pltpu.sync_copy(
                r1_v, out_hbm.at[pl.ds(wid * bpw + j * chunk, chunk)])

    return k(g1, g2, idx2.reshape(2, nw, nch, chunk))


def kernel(x, o_adj, s_adj, idx,
           W_o1, b_o1, W_s1o, b_s1o, W_s1, b_s1, W_o1s, b_o1s,
           W_o2, b_o2, W_s2o, b_s2o, Wd1, bd1, Wd2, bd2):
    ws12 = jnp.concatenate([W_s1o, W_s1], axis=1)
    wc = jnp.concatenate([W_o1s, W_o2], axis=1)
    ba = (b_o1 + b_s1o).reshape(1, NHID)
    bs1 = b_s1.reshape(1, NHID)
    bo1s = b_o1s.reshape(1, NHID)
    bo2 = b_o2.reshape(1, NHID)
    bs2o = b_s2o.reshape(1, NHID)
    wd2p = jnp.pad(Wd2, ((0, 0), (0, DPAD - RELL)))
    bd2r = jnp.pad(bd2, (0, DPAD - RELL)).reshape(1, DPAD)
    bd1r = bd1.reshape(1, NHID)

    a1, a23, wdt, wdb, bconst = _prep(
        x, ws12, W_o1, Wd1[:NHID], Wd1[NHID:], wd2p, bd1r, bd2r)
    c, t1, o8, s8 = _pass_a(o_adj, s_adj, a1, a23, ba, bs1, wc)
    d, hp = _pass_b(o8, c, t1, bo1s, bo2, W_s2o)
    g1, g2 = _pass_c(s8, d, hp, bs2o, wdt, wdb, bconst)

    out = _gather_add(g1, g2, idx.astype(jnp.int32))
    return out[:, :RELL]


# speedup vs baseline: 1.0137x; 1.0137x over previous
import functools

import jax
import jax.numpy as jnp
from jax import lax
from jax.experimental import pallas as pl
from jax.experimental.pallas import tpu as pltpu
from jax.experimental.pallas import tpu_sc as plsc

N = 10000
RPAD = 10240
NFEAT = 128
NHID = 64
RELL = 86
B = 16384
DPAD = 128

BM_A = 192
GRID_A = -(-N // BM_A)
BM = 256
GRID = RPAD // BM

F32 = jnp.float32
BF16 = jnp.bfloat16
F8 = jnp.float8_e4m3fn
SCALE = 4096.0
SCALE_RHS = 16.0
INV_SCALE = 1.0 / (SCALE * SCALE_RHS)


def _prep_body(x_ref, ws12_ref, wo1_ref, wd1t_ref, wd1b_ref, wd2_ref,
               bd1_ref, bd2_ref,
               a1_ref, a23_ref, wdt_ref, wdb_ref, bconst_ref):
    x = x_ref[...]
    a1_ref[...] = jnp.dot(x, wo1_ref[...], preferred_element_type=F32)
    a23_ref[...] = jnp.dot(x, ws12_ref[...], preferred_element_type=F32)
    wd2 = wd2_ref[...]
    wdt_ref[...] = jnp.dot(wd1t_ref[...], wd2, preferred_element_type=F32)
    wdb_ref[...] = jnp.dot(wd1b_ref[...], wd2, preferred_element_type=F32)
    bconst_ref[...] = (jnp.dot(bd1_ref[...], wd2, preferred_element_type=F32)
                       + bd2_ref[...])


def _prep(x, ws12, wo1, wd1t, wd1b, wd2p, bd1r, bd2r):
    return pl.pallas_call(
        _prep_body,
        out_shape=[
            jax.ShapeDtypeStruct((N, NHID), F32),
            jax.ShapeDtypeStruct((N, 2 * NHID), F32),
            jax.ShapeDtypeStruct((NHID, DPAD), F32),
            jax.ShapeDtypeStruct((NHID, DPAD), F32),
            jax.ShapeDtypeStruct((1, DPAD), F32),
        ],
    )(x, ws12, wo1, wd1t, wd1b, wd2p, bd1r, bd2r)


def _pass_a_body(o_ref, s_ref, a1_ref, a23_ref, ba_ref, bs1_ref, wc_ref,
                 c_ref, t1_ref, o8_ref, s8_ref):
    o = o_ref[...]
    s = s_ref[...]
    o8_ref[...] = (o * SCALE).astype(F8)
    s8_ref[...] = (s * SCALE).astype(F8)
    p = jnp.dot(o.astype(BF16), a1_ref[...].astype(BF16),
                preferred_element_type=F32)
    q = jnp.dot(s.astype(BF16), a23_ref[...].astype(BF16),
                preferred_element_type=F32)
    ox = jnp.maximum(p + q[:, :NHID] + ba_ref[...], 0.0)
    c_ref[...] = jnp.dot(ox, wc_ref[...], preferred_element_type=F32)
    t1_ref[...] = q[:, NHID:] + bs1_ref[...]


def _pass_a(o_adj, s_adj, a1, a23, ba, bs1, wc):
    adj_spec = pl.BlockSpec((BM_A, N), lambda i: (i, 0))
    row_spec = lambda w: pl.BlockSpec((BM_A, w), lambda i: (i, 0))
    full = lambda shape: pl.BlockSpec(shape, lambda i: (0, 0))
    return pl.pallas_call(
        _pass_a_body,
        grid=(GRID_A,),
        in_specs=[adj_spec, adj_spec,
                  full((N, NHID)), full((N, 2 * NHID)),
                  full((1, NHID)), full((1, NHID)), full((NHID, 2 * NHID))],
        out_specs=[row_spec(2 * NHID), row_spec(NHID),
                   adj_spec, adj_spec],
        out_shape=[jax.ShapeDtypeStruct((RPAD, 2 * NHID), F32),
                   jax.ShapeDtypeStruct((RPAD, NHID), F32),
                   jax.ShapeDtypeStruct((RPAD, N), F8),
                   jax.ShapeDtypeStruct((RPAD, N), F8)],
    )(o_adj, s_adj, a1, a23, ba, bs1, wc)


def _pass_b_body(o8_ref, c_ref, t1_ref, bo1s_ref, bo2_ref, ws2o_ref,
                 d_ref, hp_ref):
    c8 = (c_ref[...] * SCALE_RHS).astype(F8)
    r = jnp.dot(o8_ref[...], c8, preferred_element_type=F32) * INV_SCALE
    sx = jnp.maximum(t1_ref[...] + r[:, :NHID] + bo1s_ref[...], 0.0)
    d_ref[...] = jnp.dot(sx, ws2o_ref[...], preferred_element_type=F32)
    hp_ref[...] = r[:, NHID:] + bo2_ref[...]


def _pass_b(o8, c, t1, bo1s, bo2, ws2o):
    full = lambda shape: pl.BlockSpec(shape, lambda i: (0, 0))
    row_spec = lambda w: pl.BlockSpec((BM, w), lambda i: (i, 0))
    return pl.pallas_call(
        _pass_b_body,
        grid=(GRID,),
        in_specs=[pl.BlockSpec((BM, N), lambda i: (i, 0)),
                  full((N, 2 * NHID)),
                  row_spec(NHID),
                  full((1, NHID)), full((1, NHID)), full((NHID, NHID))],
        out_specs=[row_spec(NHID), row_spec(NHID)],
        out_shape=[jax.ShapeDtypeStruct((RPAD, NHID), F32),
                   jax.ShapeDtypeStruct((RPAD, NHID), F32)],
    )(o8, c, t1, bo1s, bo2, ws2o)


def _pass_c_body(s8_ref, d_ref, hp_ref, bs2o_ref, wdt_ref, wdb_ref, bconst_ref,
                 g1_ref, g2_ref):
    d8 = (d_ref[...] * SCALE_RHS).astype(F8)
    s = jnp.dot(s8_ref[...], d8, preferred_element_type=F32) * INV_SCALE
    h = hp_ref[...] + s + bs2o_ref[...]
    g1_ref[...] = jnp.dot(h, wdt_ref[...], preferred_element_type=F32) + bconst_ref[...]
    g2_ref[...] = jnp.dot(h, wdb_ref[...], preferred_element_type=F32)


def _pass_c(s8, d, hp, bs2o, wdt, wdb, bconst):
    full = lambda shape: pl.BlockSpec(shape, lambda i: (0, 0))
    row_spec = lambda w: pl.BlockSpec((BM, w), lambda i: (i, 0))
    return pl.pallas_call(
        _pass_c_body,
        grid=(GRID,),
        in_specs=[pl.BlockSpec((BM, N), lambda i: (i, 0)),
                  full((N, NHID)),
                  row_spec(NHID),
                  full((1, NHID)),
                  full((NHID, DPAD)), full((NHID, DPAD)), full((1, DPAD))],
        out_specs=[row_spec(DPAD), row_spec(DPAD)],
        out_shape=[jax.ShapeDtypeStruct((RPAD, DPAD), F32),
                   jax.ShapeDtypeStruct((RPAD, DPAD), F32)],
    )(s8, d, hp, bs2o, wdt, wdb, bconst)


def _gather_add(g1, g2, idx2):
    info = plsc.get_sparse_core_info()
    nc, ns = info.num_cores, info.num_subcores
    nw = nc * ns
    bpw = B // nw
    chunk = 128
    nch = bpw // chunk
    mesh = plsc.VectorSubcoreMesh(core_axis_name="c", subcore_axis_name="s")

    @functools.partial(
        pl.kernel, mesh=mesh,
        out_type=jax.ShapeDtypeStruct((B, DPAD), F32),
        scratch_types=[
            pltpu.VMEM((nch, chunk), jnp.int32),
            pltpu.VMEM((nch, chunk), jnp.int32),
            pltpu.VMEM((2, chunk, DPAD), F32),
            pltpu.VMEM((2, chunk, DPAD), F32),
            pltpu.SemaphoreType.DMA,
        ],
    )
    def k(g1_hbm, g2_hbm, i_hbm, out_hbm, i0_v, i1_v, r1_v, r2_v, sem):
        wid = lax.axis_index("s") * nc + lax.axis_index("c")
        pltpu.sync_copy(i_hbm.at[0, wid], i0_v)
        pltpu.sync_copy(i_hbm.at[1, wid], i1_v)
        nvec = DPAD // 16

        cps = [None, None]
        cps[0] = (pltpu.async_copy(g1_hbm.at[i0_v.at[0]], r1_v.at[0], sem),
                  pltpu.async_copy(g2_hbm.at[i1_v.at[0]], r2_v.at[0], sem))
        for j in range(nch):
            b = j % 2
            if j + 1 < nch:
                nb = (j + 1) % 2
                cps[1] = (
                    pltpu.async_copy(g1_hbm.at[i0_v.at[j + 1]], r1_v.at[nb], sem),
                    pltpu.async_copy(g2_hbm.at[i1_v.at[j + 1]], r2_v.at[nb], sem))
            cps[0][0].wait()
            cps[0][1].wait()
            cps[0] = cps[1]

            def body(r, _):
                for c in range(nvec):
                    sl = pl.ds(c * 16, 16)
                    r1_v[b, r, sl] = r1_v[b, r, sl] + r2_v[b, r, sl]
                return 0

            lax.fori_loop(0, chunk, body, 0)
            pltpu.sync_copy(
                r1_v.at[b], out_hbm.at[pl.ds(wid * bpw + j * chunk, chunk)])

    return k(g1, g2, idx2.reshape(2, nw, nch, chunk))


def kernel(x, o_adj, s_adj, idx,
           W_o1, b_o1, W_s1o, b_s1o, W_s1, b_s1, W_o1s, b_o1s,
           W_o2, b_o2, W_s2o, b_s2o, Wd1, bd1, Wd2, bd2):
    ws12 = jnp.concatenate([W_s1o, W_s1], axis=1)
    wc = jnp.concatenate([W_o1s, W_o2], axis=1)
    ba = (b_o1 + b_s1o).reshape(1, NHID)
    bs1 = b_s1.reshape(1, NHID)
    bo1s = b_o1s.reshape(1, NHID)
    bo2 = b_o2.reshape(1, NHID)
    bs2o = b_s2o.reshape(1, NHID)
    wd2p = jnp.pad(Wd2, ((0, 0), (0, DPAD - RELL)))
    bd2r = jnp.pad(bd2, (0, DPAD - RELL)).reshape(1, DPAD)
    bd1r = bd1.reshape(1, NHID)

    a1, a23, wdt, wdb, bconst = _prep(
        x, ws12, W_o1, Wd1[:NHID], Wd1[NHID:], wd2p, bd1r, bd2r)
    c, t1, o8, s8 = _pass_a(o_adj, s_adj, a1, a23, ba, bs1, wc)
    d, hp = _pass_b(o8, c, t1, bo1s, bo2, W_s2o)
    g1, g2 = _pass_c(s8, d, hp, bs2o, wdt, wdb, bconst)

    out = _gather_add(g1, g2, idx.astype(jnp.int32))
    return out[:, :RELL]
